# baseline (speedup 1.0000x reference)
import functools

import jax
import jax.numpy as jnp
from jax.experimental import pallas as pl
from jax.experimental.pallas import tpu as pltpu

LAMBDA_SOFTMAX = 9.0
LAMBDA_LSE = 6.0
MARGIN = 0.2
EPS = 1e-8


BI = 8
RP = 40


def _prep_kernel(s_ref, o_ref, t_ref):
    x = s_ref[...]
    o_ref[...] = jnp.sqrt(jnp.sum(x * x, axis=1, keepdims=True)).T
    t_ref[...] = x.astype(jnp.bfloat16).T


def _scores_kernel(R, im_ref, capT_ref, mask_ref, w1_ref, E_ref, ET_ref,
                   o_ref, imp_ref, G_ref):
    first = (pl.program_id(0) == 0) & (pl.program_id(1) == 0)

    @pl.when(first)
    def _():
        imp_ref[...] = jnp.zeros_like(imp_ref)
        G_ref[...] = jnp.zeros_like(G_ref)

    for bi in range(BI):
        imp_ref[bi * RP:bi * RP + R, :] = im_ref[bi].astype(jnp.bfloat16)
    imr = imp_ref[...]
    mask = mask_ref[...]
    raw = jnp.dot(imr, capT_ref[...], preferred_element_type=jnp.float32)
    M = BI * RP
    for bi in range(BI):
        sl = slice(bi * RP, (bi + 1) * RP)
        G_ref[sl, sl] = jax.lax.dot_general(
            imr[sl], imr[sl], (((1,), (1,)), ((), ())),
            preferred_element_type=jnp.float32)
    lk = jnp.where(raw >= 0, raw, 0.1 * raw) * mask
    nsum = jnp.dot(lk * lk, E_ref[...], preferred_element_type=jnp.float32)
    ninv = LAMBDA_SOFTMAX / (jnp.sqrt(nsum) + EPS)
    den = jnp.dot(ninv, ET_ref[...], preferred_element_type=jnp.float32)
    x = lk * den
    ri = jax.lax.broadcasted_iota(jnp.int32, (M, 1), 0)
    x = jnp.where(ri % RP < R, x, -1e30)
    e = jnp.exp(x)
    e3 = e.reshape(BI, RP, -1)
    ssum = jnp.sum(e3, axis=1)
    S12 = jnp.sum(e3 * raw.reshape(BI, RP, -1), axis=1)
    v = jnp.dot(G_ref[...], e, preferred_element_type=jnp.float32)
    S2 = jnp.sum(e3 * v.reshape(BI, RP, -1), axis=1)
    sim = S12 / jnp.maximum(w1_ref[...] * jnp.sqrt(S2), EPS * ssum)
    expd = jnp.exp(sim * LAMBDA_LSE) * mask
    ssc = jnp.dot(expd, E_ref[...], preferred_element_type=jnp.float32)
    o_ref[0, 0] = jnp.log(ssc) / LAMBDA_LSE


def _loss_kernel(sc_ref, o_ref):
    s4 = sc_ref[...]
    NJ = s4.shape[0]
    B = s4.shape[1] * s4.shape[2]
    s3 = s4.reshape(NJ, B, s4.shape[3])
    sc = jnp.concatenate([s3[j] for j in range(NJ)], axis=1)
    ri = jax.lax.broadcasted_iota(jnp.int32, (B, B), 0)
    ci = jax.lax.broadcasted_iota(jnp.int32, (B, B), 1)
    eye = ri == ci
    diag_col = jnp.sum(jnp.where(eye, sc, 0.0), axis=1, keepdims=True)
    diag_row = jnp.sum(jnp.where(eye, sc, 0.0), axis=0, keepdims=True)
    cs = jnp.maximum(MARGIN + sc - diag_col, 0.0)
    cim = jnp.maximum(MARGIN + sc - diag_row, 0.0)
    cs = jnp.where(eye, 0.0, cs)
    cim = jnp.where(eye, 0.0, cim)
    s1 = jnp.sum(jnp.max(cs, axis=1, keepdims=True), axis=0, keepdims=True)
    s2 = jnp.sum(jnp.max(cim, axis=0, keepdims=True), axis=1, keepdims=True)
    o_ref[...] = s1 + s2


@functools.partial(jax.jit, static_argnames=())
def kernel(im, im_l, s, s_l):
    B, R, D = im.shape
    W = s.shape[1]
    NJ = 2
    CJ = B // NJ
    LJ = CJ * W
    NB = B // BI

    s_flat = s.reshape(B * W, D)

    GW = 5
    w1_flat, capT = pl.pallas_call(
        _prep_kernel,
        grid=(GW,),
        in_specs=[pl.BlockSpec((B * W // GW, D), lambda g: (g, 0))],
        out_specs=[
            pl.BlockSpec((1, B * W // GW), lambda g: (0, g)),
            pl.BlockSpec((D, B * W // GW), lambda g: (0, g)),
        ],
        out_shape=[
            jax.ShapeDtypeStruct((1, B * W), jnp.float32),
            jax.ShapeDtypeStruct((D, B * W), jnp.bfloat16),
        ],
        name="word_norms_capT",
    )(s_flat)

    wpos = jnp.tile(jnp.arange(W, dtype=jnp.int32), B)
    slv = jnp.repeat(s_l.astype(jnp.int32), W)
    mask_flat = (wpos < slv).astype(jnp.float32).reshape(1, B * W)

    E = (jnp.arange(LJ, dtype=jnp.int32)[:, None] // W
         == jnp.arange(CJ, dtype=jnp.int32)[None, :]).astype(jnp.float32)
    ET = E.T

    scores4 = pl.pallas_call(
        functools.partial(_scores_kernel, R),
        grid=(NJ, NB),
        in_specs=[
            pl.BlockSpec((BI, R, D), lambda j, i: (i, 0, 0)),
            pl.BlockSpec((D, LJ), lambda j, i: (0, j)),
            pl.BlockSpec((1, LJ), lambda j, i: (0, j)),
            pl.BlockSpec((1, LJ), lambda j, i: (0, j)),
            pl.BlockSpec((LJ, CJ), lambda j, i: (0, 0)),
            pl.BlockSpec((CJ, LJ), lambda j, i: (0, 0)),
        ],
        out_specs=pl.BlockSpec((1, 1, BI, CJ), lambda j, i: (j, i, 0, 0)),
        out_shape=jax.ShapeDtypeStruct((NJ, NB, BI, CJ), jnp.float32),
        scratch_shapes=[
            pltpu.VMEM((BI * RP, D), jnp.bfloat16),
            pltpu.VMEM((BI * RP, BI * RP), jnp.float32),
        ],
        compiler_params=pltpu.CompilerParams(
            dimension_semantics=("parallel", "arbitrary"),
            vmem_limit_bytes=56 * 1024 * 1024,
        ),
        name="caption_scores",
    )(im, capT, mask_flat, w1_flat, E, ET)

    loss2 = pl.pallas_call(
        _loss_kernel,
        out_shape=jax.ShapeDtypeStruct((1, 1), jnp.float32),
        name="hinge_loss",
    )(scores4)
    return loss2.reshape(())
